# baseline (speedup 1.0000x reference)
import functools

import jax
import jax.numpy as jnp
from jax import lax
from jax.experimental import pallas as pl
from jax.experimental.pallas import tpu as pltpu
from jax.experimental.pallas import tpu_sc as plsc

N = 10000
NPAD = 10240
D = 128
E = 320000
NC, NS, L = 2, 16, 16
NW = NC * NS
K = 128
NB = 80
EPW = NB * K
EPAD = EPW * NW
CB = 8
CE = CB * K
PCE = 2048
R = 2048
ACC_ROWS = 10112
TILE_ROWS = ACC_ROWS // NS
CR = 1264


def _prep_body(x_ref, w_ref, as_ref, ad_ref, hp_ref, s_ref, d_ref):
    h = jnp.dot(x_ref[...], w_ref[...], preferred_element_type=jnp.float32)
    hp_ref[...] = h
    s_ref[...] = jnp.sum(h * as_ref[...], axis=1, keepdims=True)
    d_ref[...] = jnp.sum(h * ad_ref[...], axis=1, keepdims=True)


def _prep(x_pad, W, att_src, att_dst):
    return pl.pallas_call(
        _prep_body,
        grid=(NPAD // R,),
        in_specs=[
            pl.BlockSpec((R, D), lambda i: (i, 0)),
            pl.BlockSpec((D, D), lambda i: (0, 0)),
            pl.BlockSpec((1, D), lambda i: (0, 0)),
            pl.BlockSpec((1, D), lambda i: (0, 0)),
        ],
        out_specs=[
            pl.BlockSpec((R, D), lambda i: (i, 0)),
            pl.BlockSpec((R, 1), lambda i: (i, 0)),
            pl.BlockSpec((R, 1), lambda i: (i, 0)),
        ],
        out_shape=[
            jax.ShapeDtypeStruct((NPAD, D), jnp.float32),
            jax.ShapeDtypeStruct((NPAD, 1), jnp.float32),
            jax.ShapeDtypeStruct((NPAD, 1), jnp.float32),
        ],
    )(x_pad, W, att_src.reshape(1, D), att_dst.reshape(1, D))


def _edge_body(src_hbm, dst2_hbm, as_hbm, ad_hbm, hp_hbm,
               out_hbm, w_hbm, den_hbm, acc, sem_g0, sem_g1, sem_s0, sem_s1):
    c = lax.axis_index("c")
    s = lax.axis_index("s")
    wid = c * NS + s
    base = wid * EPW
    brow = wid * NB

    def _phase1(a_s, a_d, den, sidx_c, didx_c, w_c):
        pltpu.sync_copy(as_hbm, a_s)
        pltpu.sync_copy(ad_hbm, a_d)

        def zd(g, cy):
            den[pl.ds(g * L, L)] = jnp.zeros((L,), jnp.float32)
            return cy
        lax.fori_loop(0, ACC_ROWS // L, zd, 0)

        def chunk(t, carry):
            off = base + t * PCE
            prow = brow + t * (PCE // K)
            pltpu.sync_copy(src_hbm.at[pl.ds(off, PCE)], sidx_c)
            pltpu.sync_copy(dst2_hbm.at[pl.ds(prow, PCE // K)], didx_c)

            def grp(g, carry2):
                row = g // (K // L)
                q = lax.rem(g, K // L)
                sv = sidx_c[pl.ds(g * L, L)]
                dv = didx_c[row, pl.ds(q * L, L)]
                e = plsc.load_gather(a_s, [sv]) + plsc.load_gather(a_d, [dv])
                e = jnp.where(e >= 0.0, e, e * 0.2)
                wv = jnp.exp(e)
                gid = off + g * L + lax.iota(jnp.int32, L)
                wv = jnp.where(gid < E, wv, 0.0)
                w_c[pl.ds(g * L, L)] = wv
                plsc.addupdate_scatter(den, [dv], wv)
                return carry2
            lax.fori_loop(0, PCE // L, grp, 0)
            pltpu.sync_copy(w_c, w_hbm.at[pl.ds(off, PCE)])
            return carry
        lax.fori_loop(0, EPW // PCE, chunk, 0)
        pltpu.sync_copy(den, den_hbm.at[pl.ds(wid * ACC_ROWS, ACC_ROWS)])

    with jax.named_scope("p1_weights"):
        pl.run_scoped(_phase1,
                      pltpu.VMEM((NPAD,), jnp.float32),
                      pltpu.VMEM((NPAD,), jnp.float32),
                      pltpu.VMEM((ACC_ROWS,), jnp.float32),
                      pltpu.VMEM((PCE,), jnp.int32),
                      pltpu.VMEM((PCE // K, K), jnp.int32),
                      pltpu.VMEM((PCE,), jnp.float32))

    def _phase2(rows0, rows1, schunk, dchunk, wbuf, dst0, dst1, wstage):
        t0 = s * TILE_ROWS
        rem = TILE_ROWS % K

        def zr(k, cy):
            for j in range(D // L):
                rows0[k, pl.ds(j * L, L)] = jnp.zeros((L,), jnp.float32)
            return cy
        lax.fori_loop(0, K, zr, 0)
        for r in range(TILE_ROWS // K):
            pltpu.sync_copy(rows0, acc.at[pl.ds(t0 + r * K, K)])
        pltpu.sync_copy(rows0.at[pl.ds(0, rem)],
                        acc.at[pl.ds(t0 + (TILE_ROWS // K) * K, rem)])
        plsc.subcore_barrier()

        def load_chunk(t):
            pltpu.sync_copy(src_hbm.at[pl.ds(base + t * CE, CE)], schunk)
            pltpu.sync_copy(dst2_hbm.at[pl.ds(brow + t * CB, CB)], dchunk)
            pltpu.sync_copy(w_hbm.at[pl.ds(base + t * CE, CE)],
                            wbuf.at[pl.ds(0, CE)])

        def stage(i, dstg):
            j = lax.rem(i, CB)
            for q in range(K // L):
                dstg[0, pl.ds(q * L, L)] = dchunk[j, pl.ds(q * L, L)]
                wstage[pl.ds(q * L, L)] = wbuf[pl.ds(j * K + q * L, L)]

        def scale(rows):
            @plsc.parallel_loop(0, K, unroll=4)
            def sc(k):
                wk = wstage[pl.ds(k, L)][0]
                for j in range(D // L):
                    rows[k, pl.ds(j * L, L)] = rows[k, pl.ds(j * L, L)] * wk

        def _pipeline():
            load_chunk(0)
            pltpu.async_copy(hp_hbm.at[schunk.at[pl.ds(0, K)]], rows0,
                             sem_g0)

            def piter(m, cy):
                i0 = m * 2
                i1 = i0 + 1
                stage(i0, dst0)

                @pl.when(m >= 1)
                def _():
                    pltpu.make_async_copy(rows1, acc.at[dst1.at[0]],
                                          sem_s1).wait()
                j1 = lax.rem(i1, CB)
                pltpu.async_copy(hp_hbm.at[schunk.at[pl.ds(j1 * K, K)]],
                                 rows1, sem_g1)
                pltpu.make_async_copy(hp_hbm.at[schunk.at[pl.ds(0, K)]],
                                      rows0, sem_g0).wait()
                scale(rows0)
                pltpu.async_copy(rows0, acc.at[dst0.at[0]], sem_s0, add=True)
                stage(i1, dst1)

                @pl.when(m < NB // 2 - 1)
                def _():
                    @pl.when(lax.rem(i1 + 1, CB) == 0)
                    def _():
                        load_chunk((i1 + 1) // CB)
                    pltpu.make_async_copy(rows0, acc.at[dst0.at[0]],
                                          sem_s0).wait()
                    j2 = lax.rem(i1 + 1, CB)
                    pltpu.async_copy(hp_hbm.at[schunk.at[pl.ds(j2 * K, K)]],
                                     rows0, sem_g0)
                pltpu.make_async_copy(hp_hbm.at[schunk.at[pl.ds(0, K)]],
                                      rows1, sem_g1).wait()
                scale(rows1)
                pltpu.async_copy(rows1, acc.at[dst1.at[0]], sem_s1, add=True)
                return cy
            lax.fori_loop(0, NB // 2, piter, 0)

            pltpu.make_async_copy(rows0, acc.at[dst0.at[0]], sem_s0).wait()
            pltpu.make_async_copy(rows1, acc.at[dst1.at[0]], sem_s1).wait()

        with jax.named_scope("p2_pipeline"):
            _pipeline()
        plsc.subcore_barrier()
        for r in range(TILE_ROWS // K):
            row0 = t0 + r * K
            pltpu.sync_copy(acc.at[pl.ds(row0, K)],
                            out_hbm.at[c, pl.ds(row0, K)])
        row0 = t0 + (TILE_ROWS // K) * K
        pltpu.sync_copy(acc.at[pl.ds(row0, rem)],
                        out_hbm.at[c, pl.ds(row0, rem)])

    pl.run_scoped(_phase2,
                  pltpu.VMEM((K, D), jnp.float32),
                  pltpu.VMEM((K, D), jnp.float32),
                  pltpu.VMEM((CE,), jnp.int32),
                  pltpu.VMEM((CB, K), jnp.int32),
                  pltpu.VMEM((CE + L,), jnp.float32),
                  pltpu.VMEM((1, K), jnp.int32),
                  pltpu.VMEM((1, K), jnp.int32),
                  pltpu.VMEM((K + L,), jnp.float32))


def _edges(src_pad, dst2d, a_s, a_d, hp):
    mesh = plsc.VectorSubcoreMesh(
        core_axis_name="c", subcore_axis_name="s",
        num_cores=NC, num_subcores=NS)
    k = functools.partial(
        pl.kernel,
        out_type=(jax.ShapeDtypeStruct((NC, ACC_ROWS, D), jnp.float32),
                  jax.ShapeDtypeStruct((EPAD,), jnp.float32),
                  jax.ShapeDtypeStruct((NW * ACC_ROWS,), jnp.float32)),
        mesh=mesh,
        compiler_params=pltpu.CompilerParams(
            needs_layout_passes=False, use_tc_tiling_on_sc=True),
        scratch_types=[
            pltpu.VMEM_SHARED((ACC_ROWS, D), jnp.float32),
            pltpu.SemaphoreType.DMA,
            pltpu.SemaphoreType.DMA,
            pltpu.SemaphoreType.DMA,
            pltpu.SemaphoreType.DMA,
        ],
    )(_edge_body)
    return k(src_pad, dst2d, a_s, a_d, hp)


def _combine_body(p_ref, den_ref, hp_ref, as_ref, ad_ref, b_ref, out_ref):
    h = hp_ref[...]
    e = (jnp.sum(h * as_ref[...], axis=1, keepdims=True)
         + jnp.sum(h * ad_ref[...], axis=1, keepdims=True))
    wself = jnp.exp(jnp.where(e >= 0.0, e, e * 0.2))
    num = p_ref[0] + p_ref[1] + wself * h
    den_col = jnp.sum(den_ref[...], axis=1, keepdims=True)
    out_ref[...] = num / (den_col + wself + 1e-16) + b_ref[...]


def _combine(p, den, hp, att_src, att_dst, bias):
    return pl.pallas_call(
        _combine_body,
        grid=(ACC_ROWS // CR,),
        in_specs=[
            pl.BlockSpec((NC, CR, D), lambda i: (0, i, 0)),
            pl.BlockSpec((CR, NW), lambda i: (i, 0)),
            pl.BlockSpec((CR, D), lambda i: (i, 0)),
            pl.BlockSpec((1, D), lambda i: (0, 0)),
            pl.BlockSpec((1, D), lambda i: (0, 0)),
            pl.BlockSpec((1, D), lambda i: (0, 0)),
        ],
        out_specs=pl.BlockSpec((CR, D), lambda i: (i, 0)),
        out_shape=jax.ShapeDtypeStruct((ACC_ROWS, D), jnp.float32),
    )(p, den, hp, att_src.reshape(1, D), att_dst.reshape(1, D),
      bias.reshape(1, D))


def kernel(x, edge_index, W, att_src, att_dst, bias):
    src = edge_index[0].astype(jnp.int32)
    dst = edge_index[1].astype(jnp.int32)
    spread = (jnp.arange(EPAD - E, dtype=jnp.int32) * 37) % N
    src_pad = jnp.concatenate([src, spread])
    dst_pad = jnp.concatenate([dst, spread])
    dst2d = dst_pad.reshape(EPAD // K, K)
    x_pad = jnp.pad(x, ((0, NPAD - N), (0, 0)))
    hp, a_s, a_d = _prep(x_pad, W, att_src, att_dst)
    p, _, den = _edges(src_pad, dst2d,
                       a_s.reshape(NPAD), a_d.reshape(NPAD), hp)
    den_t = den.reshape(NW, ACC_ROWS).T
    out = _combine(p, den_t, hp, att_src, att_dst, bias)
    return out[:N]


# speedup vs baseline: 1.0151x; 1.0151x over previous
import functools

import jax
import jax.numpy as jnp
from jax import lax
from jax.experimental import pallas as pl
from jax.experimental.pallas import tpu as pltpu
from jax.experimental.pallas import tpu_sc as plsc

N = 10000
NPAD = 10240
D = 128
E = 320000
NC, NS, L = 2, 16, 16
NW = NC * NS
K = 128
NB = 80
EPW = NB * K
EPAD = EPW * NW
CB = 8
CE = CB * K
PCE = 2048
R = 2048
ACC_ROWS = 10112
TILE_ROWS = ACC_ROWS // NS
CR = 1000


def _prep_body(x_ref, w_ref, as_ref, ad_ref, hp_ref, s_ref, d_ref):
    h = jnp.dot(x_ref[...], w_ref[...], preferred_element_type=jnp.float32)
    hp_ref[...] = h
    s_ref[...] = jnp.sum(h * as_ref[...], axis=1, keepdims=True)
    d_ref[...] = jnp.sum(h * ad_ref[...], axis=1, keepdims=True)


def _prep(x_pad, W, att_src, att_dst):
    return pl.pallas_call(
        _prep_body,
        grid=(NPAD // R,),
        in_specs=[
            pl.BlockSpec((R, D), lambda i: (i, 0)),
            pl.BlockSpec((D, D), lambda i: (0, 0)),
            pl.BlockSpec((1, D), lambda i: (0, 0)),
            pl.BlockSpec((1, D), lambda i: (0, 0)),
        ],
        out_specs=[
            pl.BlockSpec((R, D), lambda i: (i, 0)),
            pl.BlockSpec((R, 1), lambda i: (i, 0)),
            pl.BlockSpec((R, 1), lambda i: (i, 0)),
        ],
        out_shape=[
            jax.ShapeDtypeStruct((NPAD, D), jnp.float32),
            jax.ShapeDtypeStruct((NPAD, 1), jnp.float32),
            jax.ShapeDtypeStruct((NPAD, 1), jnp.float32),
        ],
    )(x_pad, W, att_src.reshape(1, D), att_dst.reshape(1, D))


def _edge_body(src_hbm, dst2_hbm, as_hbm, ad_hbm, hp_hbm,
               out_hbm, w_hbm, den_hbm, acc, sem_g0, sem_g1, sem_s0, sem_s1):
    c = lax.axis_index("c")
    s = lax.axis_index("s")
    wid = c * NS + s
    base = wid * EPW
    brow = wid * NB

    def _phase1(a_s, a_d, den, sidx_c, didx_c, w_c):
        pltpu.sync_copy(as_hbm, a_s)
        pltpu.sync_copy(ad_hbm, a_d)

        def zd(g, cy):
            den[pl.ds(g * L, L)] = jnp.zeros((L,), jnp.float32)
            return cy
        lax.fori_loop(0, ACC_ROWS // L, zd, 0)

        def chunk(t, carry):
            off = base + t * PCE
            prow = brow + t * (PCE // K)
            pltpu.sync_copy(src_hbm.at[pl.ds(off, PCE)], sidx_c)
            pltpu.sync_copy(dst2_hbm.at[pl.ds(prow, PCE // K)], didx_c)

            @plsc.parallel_loop(0, PCE // L, unroll=2)
            def grp(g):
                row = g // (K // L)
                q = lax.rem(g, K // L)
                sv = sidx_c[pl.ds(g * L, L)]
                dv = didx_c[row, pl.ds(q * L, L)]
                e = plsc.load_gather(a_s, [sv]) + plsc.load_gather(a_d, [dv])
                e = jnp.where(e >= 0.0, e, e * 0.2)
                wv = jnp.exp(e)
                gid = off + g * L + lax.iota(jnp.int32, L)
                wv = jnp.where(gid < E, wv, 0.0)
                w_c[pl.ds(g * L, L)] = wv
                plsc.addupdate_scatter(den, [dv], wv)
            pltpu.sync_copy(w_c, w_hbm.at[pl.ds(off, PCE)])
            return carry
        lax.fori_loop(0, EPW // PCE, chunk, 0)
        pltpu.sync_copy(den, den_hbm.at[pl.ds(wid * ACC_ROWS, ACC_ROWS)])

    with jax.named_scope("p1_weights"):
        pl.run_scoped(_phase1,
                      pltpu.VMEM((NPAD,), jnp.float32),
                      pltpu.VMEM((NPAD,), jnp.float32),
                      pltpu.VMEM((ACC_ROWS,), jnp.float32),
                      pltpu.VMEM((PCE,), jnp.int32),
                      pltpu.VMEM((PCE // K, K), jnp.int32),
                      pltpu.VMEM((PCE,), jnp.float32))

    def _phase2(rows0, rows1, schunk, dchunk, wbuf, dst0, dst1, wstage):
        t0 = s * TILE_ROWS
        rem = TILE_ROWS % K

        def zr(k, cy):
            for j in range(D // L):
                rows0[k, pl.ds(j * L, L)] = jnp.zeros((L,), jnp.float32)
            return cy
        lax.fori_loop(0, K, zr, 0)
        for r in range(TILE_ROWS // K):
            pltpu.sync_copy(rows0, acc.at[pl.ds(t0 + r * K, K)])
        pltpu.sync_copy(rows0.at[pl.ds(0, rem)],
                        acc.at[pl.ds(t0 + (TILE_ROWS // K) * K, rem)])
        plsc.subcore_barrier()

        def load_chunk(t):
            pltpu.sync_copy(src_hbm.at[pl.ds(base + t * CE, CE)], schunk)
            pltpu.sync_copy(dst2_hbm.at[pl.ds(brow + t * CB, CB)], dchunk)
            pltpu.sync_copy(w_hbm.at[pl.ds(base + t * CE, CE)],
                            wbuf.at[pl.ds(0, CE)])

        def stage(i, dstg):
            j = lax.rem(i, CB)
            for q in range(K // L):
                dstg[0, pl.ds(q * L, L)] = dchunk[j, pl.ds(q * L, L)]
                wstage[pl.ds(q * L, L)] = wbuf[pl.ds(j * K + q * L, L)]

        def scale(rows):
            @plsc.parallel_loop(0, K, unroll=4)
            def sc(k):
                wk = wstage[pl.ds(k, L)][0]
                for j in range(D // L):
                    rows[k, pl.ds(j * L, L)] = rows[k, pl.ds(j * L, L)] * wk

        def _pipeline():
            load_chunk(0)
            pltpu.async_copy(hp_hbm.at[schunk.at[pl.ds(0, K)]], rows0,
                             sem_g0)

            def piter(m, cy):
                i0 = m * 2
                i1 = i0 + 1
                stage(i0, dst0)

                @pl.when(m >= 1)
                def _():
                    pltpu.make_async_copy(rows1, acc.at[dst1.at[0]],
                                          sem_s1).wait()
                j1 = lax.rem(i1, CB)
                pltpu.async_copy(hp_hbm.at[schunk.at[pl.ds(j1 * K, K)]],
                                 rows1, sem_g1)
                pltpu.make_async_copy(hp_hbm.at[schunk.at[pl.ds(0, K)]],
                                      rows0, sem_g0).wait()
                scale(rows0)
                pltpu.async_copy(rows0, acc.at[dst0.at[0]], sem_s0, add=True)
                stage(i1, dst1)

                @pl.when(m < NB // 2 - 1)
                def _():
                    @pl.when(lax.rem(i1 + 1, CB) == 0)
                    def _():
                        load_chunk((i1 + 1) // CB)
                    pltpu.make_async_copy(rows0, acc.at[dst0.at[0]],
                                          sem_s0).wait()
                    j2 = lax.rem(i1 + 1, CB)
                    pltpu.async_copy(hp_hbm.at[schunk.at[pl.ds(j2 * K, K)]],
                                     rows0, sem_g0)
                pltpu.make_async_copy(hp_hbm.at[schunk.at[pl.ds(0, K)]],
                                      rows1, sem_g1).wait()
                scale(rows1)
                pltpu.async_copy(rows1, acc.at[dst1.at[0]], sem_s1, add=True)
                return cy
            lax.fori_loop(0, NB // 2, piter, 0)

            pltpu.make_async_copy(rows0, acc.at[dst0.at[0]], sem_s0).wait()
            pltpu.make_async_copy(rows1, acc.at[dst1.at[0]], sem_s1).wait()

        with jax.named_scope("p2_pipeline"):
            _pipeline()
        plsc.subcore_barrier()
        for r in range(TILE_ROWS // K):
            row0 = t0 + r * K
            pltpu.sync_copy(acc.at[pl.ds(row0, K)],
                            out_hbm.at[c, pl.ds(row0, K)])
        row0 = t0 + (TILE_ROWS // K) * K
        pltpu.sync_copy(acc.at[pl.ds(row0, rem)],
                        out_hbm.at[c, pl.ds(row0, rem)])

    pl.run_scoped(_phase2,
                  pltpu.VMEM((K, D), jnp.float32),
                  pltpu.VMEM((K, D), jnp.float32),
                  pltpu.VMEM((CE,), jnp.int32),
                  pltpu.VMEM((CB, K), jnp.int32),
                  pltpu.VMEM((CE + L,), jnp.float32),
                  pltpu.VMEM((1, K), jnp.int32),
                  pltpu.VMEM((1, K), jnp.int32),
                  pltpu.VMEM((K + L,), jnp.float32))


def _edges(src_pad, dst2d, a_s, a_d, hp):
    mesh = plsc.VectorSubcoreMesh(
        core_axis_name="c", subcore_axis_name="s",
        num_cores=NC, num_subcores=NS)
    k = functools.partial(
        pl.kernel,
        out_type=(jax.ShapeDtypeStruct((NC, ACC_ROWS, D), jnp.float32),
                  jax.ShapeDtypeStruct((EPAD,), jnp.float32),
                  jax.ShapeDtypeStruct((NW * ACC_ROWS,), jnp.float32)),
        mesh=mesh,
        compiler_params=pltpu.CompilerParams(
            needs_layout_passes=False, use_tc_tiling_on_sc=True),
        scratch_types=[
            pltpu.VMEM_SHARED((ACC_ROWS, D), jnp.float32),
            pltpu.SemaphoreType.DMA,
            pltpu.SemaphoreType.DMA,
            pltpu.SemaphoreType.DMA,
            pltpu.SemaphoreType.DMA,
        ],
    )(_edge_body)
    return k(src_pad, dst2d, a_s, a_d, hp)


def _combine_body(p_ref, den_ref, hp_ref, as_ref, ad_ref, b_ref, out_ref):
    h = hp_ref[...]
    e = (jnp.sum(h * as_ref[...], axis=1, keepdims=True)
         + jnp.sum(h * ad_ref[...], axis=1, keepdims=True))
    wself = jnp.exp(jnp.where(e >= 0.0, e, e * 0.2))
    num = p_ref[0] + p_ref[1] + wself * h
    den_col = jnp.sum(den_ref[...], axis=1, keepdims=True)
    out_ref[...] = num / (den_col + wself + 1e-16) + b_ref[...]


def _combine(p, den, hp, att_src, att_dst, bias):
    return pl.pallas_call(
        _combine_body,
        grid=(N // CR,),
        in_specs=[
            pl.BlockSpec((NC, CR, D), lambda i: (0, i, 0)),
            pl.BlockSpec((CR, NW), lambda i: (i, 0)),
            pl.BlockSpec((CR, D), lambda i: (i, 0)),
            pl.BlockSpec((1, D), lambda i: (0, 0)),
            pl.BlockSpec((1, D), lambda i: (0, 0)),
            pl.BlockSpec((1, D), lambda i: (0, 0)),
        ],
        out_specs=pl.BlockSpec((CR, D), lambda i: (i, 0)),
        out_shape=jax.ShapeDtypeStruct((N, D), jnp.float32),
    )(p, den, hp, att_src.reshape(1, D), att_dst.reshape(1, D),
      bias.reshape(1, D))


def kernel(x, edge_index, W, att_src, att_dst, bias):
    src = edge_index[0].astype(jnp.int32)
    dst = edge_index[1].astype(jnp.int32)
    spread = (jnp.arange(EPAD - E, dtype=jnp.int32) * 37) % N
    src_pad = jnp.concatenate([src, spread])
    dst_pad = jnp.concatenate([dst, spread])
    dst2d = dst_pad.reshape(EPAD // K, K)
    x_pad = jnp.pad(x, ((0, NPAD - N), (0, 0)))
    hp, a_s, a_d = _prep(x_pad, W, att_src, att_dst)
    p, _, den = _edges(src_pad, dst2d,
                       a_s.reshape(NPAD), a_d.reshape(NPAD), hp)
    den_t = den.reshape(NW, ACC_ROWS).T
    return _combine(p, den_t, hp, att_src, att_dst, bias)


# speedup vs baseline: 1.0355x; 1.0201x over previous
import functools

import jax
import jax.numpy as jnp
from jax import lax
from jax.experimental import pallas as pl
from jax.experimental.pallas import tpu as pltpu
from jax.experimental.pallas import tpu_sc as plsc

N = 10000
NPAD = 10240
D = 128
E = 320000
NC, NS, L = 2, 16, 16
NW = NC * NS
K = 128
NB = 80
EPW = NB * K
EPAD = EPW * NW
CB = 16
CE = CB * K
PCE = 2048
R = 2048
ACC_ROWS = 10112
TILE_ROWS = ACC_ROWS // NS
CR = 1000


def _prep_body(x_ref, w_ref, as_ref, ad_ref, hp_ref, s_ref, d_ref):
    h = jnp.dot(x_ref[...], w_ref[...], preferred_element_type=jnp.float32)
    hp_ref[...] = h
    s_ref[...] = jnp.sum(h * as_ref[...], axis=1, keepdims=True)
    d_ref[...] = jnp.sum(h * ad_ref[...], axis=1, keepdims=True)


def _prep(x_pad, W, att_src, att_dst):
    return pl.pallas_call(
        _prep_body,
        grid=(NPAD // R,),
        in_specs=[
            pl.BlockSpec((R, D), lambda i: (i, 0)),
            pl.BlockSpec((D, D), lambda i: (0, 0)),
            pl.BlockSpec((1, D), lambda i: (0, 0)),
            pl.BlockSpec((1, D), lambda i: (0, 0)),
        ],
        out_specs=[
            pl.BlockSpec((R, D), lambda i: (i, 0)),
            pl.BlockSpec((R, 1), lambda i: (i, 0)),
            pl.BlockSpec((R, 1), lambda i: (i, 0)),
        ],
        out_shape=[
            jax.ShapeDtypeStruct((NPAD, D), jnp.float32),
            jax.ShapeDtypeStruct((NPAD, 1), jnp.float32),
            jax.ShapeDtypeStruct((NPAD, 1), jnp.float32),
        ],
    )(x_pad, W, att_src.reshape(1, D), att_dst.reshape(1, D))


def _edge_body(src_hbm, dst2_hbm, as_hbm, ad_hbm, hp_hbm,
               out_hbm, w_hbm, den_hbm, acc, sem_g0, sem_g1, sem_s0, sem_s1):
    c = lax.axis_index("c")
    s = lax.axis_index("s")
    wid = c * NS + s
    base = wid * EPW
    brow = wid * NB

    def _phase1(a_s, a_d, den, sidx_c, didx_c, w_c):
        pltpu.sync_copy(as_hbm, a_s)
        pltpu.sync_copy(ad_hbm, a_d)

        def zd(g, cy):
            den[pl.ds(g * L, L)] = jnp.zeros((L,), jnp.float32)
            return cy
        lax.fori_loop(0, ACC_ROWS // L, zd, 0)

        def chunk(t, carry):
            off = base + t * PCE
            prow = brow + t * (PCE // K)
            pltpu.sync_copy(src_hbm.at[pl.ds(off, PCE)], sidx_c)
            pltpu.sync_copy(dst2_hbm.at[pl.ds(prow, PCE // K)], didx_c)

            @plsc.parallel_loop(0, PCE // L, unroll=2)
            def grp(g):
                row = g // (K // L)
                q = lax.rem(g, K // L)
                sv = sidx_c[pl.ds(g * L, L)]
                dv = didx_c[row, pl.ds(q * L, L)]
                e = plsc.load_gather(a_s, [sv]) + plsc.load_gather(a_d, [dv])
                e = jnp.where(e >= 0.0, e, e * 0.2)
                wv = jnp.exp(e)
                gid = off + g * L + lax.iota(jnp.int32, L)
                wv = jnp.where(gid < E, wv, 0.0)
                w_c[pl.ds(g * L, L)] = wv
                plsc.addupdate_scatter(den, [dv], wv)
            pltpu.sync_copy(w_c, w_hbm.at[pl.ds(off, PCE)])
            return carry
        lax.fori_loop(0, EPW // PCE, chunk, 0)
        pltpu.sync_copy(den, den_hbm.at[pl.ds(wid * ACC_ROWS, ACC_ROWS)])

    with jax.named_scope("p1_weights"):
        pl.run_scoped(_phase1,
                      pltpu.VMEM((NPAD,), jnp.float32),
                      pltpu.VMEM((NPAD,), jnp.float32),
                      pltpu.VMEM((ACC_ROWS,), jnp.float32),
                      pltpu.VMEM((PCE,), jnp.int32),
                      pltpu.VMEM((PCE // K, K), jnp.int32),
                      pltpu.VMEM((PCE,), jnp.float32))

    def _phase2(rows0, rows1, schunk, dchunk, wbuf, dst0, dst1, wstage):
        t0 = s * TILE_ROWS
        rem = TILE_ROWS % K

        def zr(k, cy):
            for j in range(D // L):
                rows0[k, pl.ds(j * L, L)] = jnp.zeros((L,), jnp.float32)
            return cy
        lax.fori_loop(0, K, zr, 0)
        for r in range(TILE_ROWS // K):
            pltpu.sync_copy(rows0, acc.at[pl.ds(t0 + r * K, K)])
        pltpu.sync_copy(rows0.at[pl.ds(0, rem)],
                        acc.at[pl.ds(t0 + (TILE_ROWS // K) * K, rem)])
        plsc.subcore_barrier()

        def load_chunk(t):
            pltpu.sync_copy(src_hbm.at[pl.ds(base + t * CE, CE)], schunk)
            pltpu.sync_copy(dst2_hbm.at[pl.ds(brow + t * CB, CB)], dchunk)
            pltpu.sync_copy(w_hbm.at[pl.ds(base + t * CE, CE)],
                            wbuf.at[pl.ds(0, CE)])

        def stage(i, dstg):
            j = lax.rem(i, CB)
            for q in range(K // L):
                dstg[0, pl.ds(q * L, L)] = dchunk[j, pl.ds(q * L, L)]
                wstage[pl.ds(q * L, L)] = wbuf[pl.ds(j * K + q * L, L)]

        def scale(rows):
            @plsc.parallel_loop(0, K, unroll=4)
            def sc(k):
                wk = wstage[pl.ds(k, L)][0]
                for j in range(D // L):
                    rows[k, pl.ds(j * L, L)] = rows[k, pl.ds(j * L, L)] * wk

        def _pipeline():
            load_chunk(0)
            pltpu.async_copy(hp_hbm.at[schunk.at[pl.ds(0, K)]], rows0,
                             sem_g0)

            def piter(m, cy):
                i0 = m * 2
                i1 = i0 + 1
                stage(i0, dst0)

                @pl.when(m >= 1)
                def _():
                    pltpu.make_async_copy(rows1, acc.at[dst1.at[0]],
                                          sem_s1).wait()
                j1 = lax.rem(i1, CB)
                pltpu.async_copy(hp_hbm.at[schunk.at[pl.ds(j1 * K, K)]],
                                 rows1, sem_g1)
                pltpu.make_async_copy(hp_hbm.at[schunk.at[pl.ds(0, K)]],
                                      rows0, sem_g0).wait()
                scale(rows0)
                pltpu.async_copy(rows0, acc.at[dst0.at[0]], sem_s0, add=True)
                stage(i1, dst1)

                @pl.when(m < NB // 2 - 1)
                def _():
                    @pl.when(lax.rem(i1 + 1, CB) == 0)
                    def _():
                        load_chunk((i1 + 1) // CB)
                    pltpu.make_async_copy(rows0, acc.at[dst0.at[0]],
                                          sem_s0).wait()
                    j2 = lax.rem(i1 + 1, CB)
                    pltpu.async_copy(hp_hbm.at[schunk.at[pl.ds(j2 * K, K)]],
                                     rows0, sem_g0)
                pltpu.make_async_copy(hp_hbm.at[schunk.at[pl.ds(0, K)]],
                                      rows1, sem_g1).wait()
                scale(rows1)
                pltpu.async_copy(rows1, acc.at[dst1.at[0]], sem_s1, add=True)
                return cy
            lax.fori_loop(0, NB // 2, piter, 0)

            pltpu.make_async_copy(rows0, acc.at[dst0.at[0]], sem_s0).wait()
            pltpu.make_async_copy(rows1, acc.at[dst1.at[0]], sem_s1).wait()

        with jax.named_scope("p2_pipeline"):
            _pipeline()
        plsc.subcore_barrier()
        for r in range(TILE_ROWS // K):
            row0 = t0 + r * K
            pltpu.sync_copy(acc.at[pl.ds(row0, K)],
                            out_hbm.at[c, pl.ds(row0, K)])
        row0 = t0 + (TILE_ROWS // K) * K
        pltpu.sync_copy(acc.at[pl.ds(row0, rem)],
                        out_hbm.at[c, pl.ds(row0, rem)])

    pl.run_scoped(_phase2,
                  pltpu.VMEM((K, D), jnp.float32),
                  pltpu.VMEM((K, D), jnp.float32),
                  pltpu.VMEM((CE,), jnp.int32),
                  pltpu.VMEM((CB, K), jnp.int32),
                  pltpu.VMEM((CE + L,), jnp.float32),
                  pltpu.VMEM((1, K), jnp.int32),
                  pltpu.VMEM((1, K), jnp.int32),
                  pltpu.VMEM((K + L,), jnp.float32))


def _edges(src_pad, dst2d, a_s, a_d, hp):
    mesh = plsc.VectorSubcoreMesh(
        core_axis_name="c", subcore_axis_name="s",
        num_cores=NC, num_subcores=NS)
    k = functools.partial(
        pl.kernel,
        out_type=(jax.ShapeDtypeStruct((NC, ACC_ROWS, D), jnp.float32),
                  jax.ShapeDtypeStruct((EPAD,), jnp.float32),
                  jax.ShapeDtypeStruct((NW * ACC_ROWS,), jnp.float32)),
        mesh=mesh,
        compiler_params=pltpu.CompilerParams(
            needs_layout_passes=False, use_tc_tiling_on_sc=True),
        scratch_types=[
            pltpu.VMEM_SHARED((ACC_ROWS, D), jnp.float32),
            pltpu.SemaphoreType.DMA,
            pltpu.SemaphoreType.DMA,
            pltpu.SemaphoreType.DMA,
            pltpu.SemaphoreType.DMA,
        ],
    )(_edge_body)
    return k(src_pad, dst2d, a_s, a_d, hp)


def _combine_body(p_ref, den_ref, hp_ref, as_ref, ad_ref, b_ref, out_ref):
    h = hp_ref[...]
    e = (jnp.sum(h * as_ref[...], axis=1, keepdims=True)
         + jnp.sum(h * ad_ref[...], axis=1, keepdims=True))
    wself = jnp.exp(jnp.where(e >= 0.0, e, e * 0.2))
    num = p_ref[0] + p_ref[1] + wself * h
    den_col = jnp.sum(den_ref[...], axis=1, keepdims=True)
    out_ref[...] = num / (den_col + wself + 1e-16) + b_ref[...]


def _combine(p, den, hp, att_src, att_dst, bias):
    return pl.pallas_call(
        _combine_body,
        grid=(N // CR,),
        in_specs=[
            pl.BlockSpec((NC, CR, D), lambda i: (0, i, 0)),
            pl.BlockSpec((CR, NW), lambda i: (i, 0)),
            pl.BlockSpec((CR, D), lambda i: (i, 0)),
            pl.BlockSpec((1, D), lambda i: (0, 0)),
            pl.BlockSpec((1, D), lambda i: (0, 0)),
            pl.BlockSpec((1, D), lambda i: (0, 0)),
        ],
        out_specs=pl.BlockSpec((CR, D), lambda i: (i, 0)),
        out_shape=jax.ShapeDtypeStruct((N, D), jnp.float32),
    )(p, den, hp, att_src.reshape(1, D), att_dst.reshape(1, D),
      bias.reshape(1, D))


def kernel(x, edge_index, W, att_src, att_dst, bias):
    src = edge_index[0].astype(jnp.int32)
    dst = edge_index[1].astype(jnp.int32)
    spread = (jnp.arange(EPAD - E, dtype=jnp.int32) * 37) % N
    src_pad = jnp.concatenate([src, spread])
    dst_pad = jnp.concatenate([dst, spread])
    dst2d = dst_pad.reshape(EPAD // K, K)
    x_pad = jnp.pad(x, ((0, NPAD - N), (0, 0)))
    hp, a_s, a_d = _prep(x_pad, W, att_src, att_dst)
    p, _, den = _edges(src_pad, dst2d,
                       a_s.reshape(NPAD), a_d.reshape(NPAD), hp)
    den_t = den.reshape(NW, ACC_ROWS).T
    return _combine(p, den_t, hp, att_src, att_dst, bias)
